# baseline (speedup 1.0000x reference)
import jax
import jax.numpy as jnp
from jax.experimental import pallas as pl

BN_EPS = 1e-5
COS_EPS = 1e-8
BLOCK_E = 4096


def _pae_block(x1_ref, x2_ref, w1a_ref, w2t_ref, b2t_ref, o_ref):
    blk = x1_ref.shape[1]
    xx = jnp.concatenate([x1_ref[...], x2_ref[...]], axis=1)
    a = jnp.dot(w1a_ref[...], xx, preferred_element_type=jnp.float32)
    ab = jnp.maximum(a.astype(jnp.bfloat16), jnp.bfloat16(0))
    h = jnp.dot(w2t_ref[...], ab, preferred_element_type=jnp.float32) + b2t_ref[...]
    h1 = h[:, :blk]
    h2 = h[:, blk:]
    s11 = jnp.sum(h1 * h1, axis=0)
    s22 = jnp.sum(h2 * h2, axis=0)
    s12 = jnp.sum(h1 * h2, axis=0)
    n1 = jnp.maximum(jnp.sqrt(s11), COS_EPS)
    n2 = jnp.maximum(jnp.sqrt(s22), COS_EPS)
    o_ref[...] = (s12 / (n1 * n2) + 1.0) * 0.5


def kernel(edge_index, edgenet_input, flag, W1, b1, gamma, beta,
           running_mean, running_var, W2, b2):
    n_edges, feat = edgenet_input.shape
    in_dim = feat // 2
    hidden = W1.shape[1]
    nblk = pl.cdiv(n_edges, BLOCK_E)
    epad = nblk * BLOCK_E

    xb = edgenet_input.astype(jnp.bfloat16)
    ones = jnp.ones((1, n_edges), jnp.bfloat16)
    pad = jnp.zeros((in_dim + 1, epad - n_edges), jnp.bfloat16)
    half1 = jnp.concatenate([xb[:, :in_dim].T, ones], axis=0)
    half2 = jnp.concatenate([xb[:, in_dim:].T, ones], axis=0)
    xall = jnp.concatenate([half1, pad, half2, pad], axis=1)

    scale = gamma * jax.lax.rsqrt(running_var + BN_EPS)
    w1a = jnp.concatenate([W1.T, b1[:, None]], axis=1).astype(jnp.bfloat16)
    w2t = (W2 * scale[:, None]).T.astype(jnp.bfloat16)
    b2f = b2 + (beta - running_mean * scale) @ W2

    edge_weight = pl.pallas_call(
        _pae_block,
        grid=(nblk,),
        in_specs=[
            pl.BlockSpec((in_dim + 1, BLOCK_E), lambda i: (0, i)),
            pl.BlockSpec((in_dim + 1, BLOCK_E), lambda i, n=nblk: (0, i + n)),
            pl.BlockSpec((hidden, in_dim + 1), lambda i: (0, 0)),
            pl.BlockSpec((hidden, hidden), lambda i: (0, 0)),
            pl.BlockSpec((hidden, 1), lambda i: (0, 0)),
        ],
        out_specs=pl.BlockSpec((BLOCK_E,), lambda i: (i,)),
        out_shape=jax.ShapeDtypeStruct((n_edges,), jnp.float32),
    )(xall, xall, w1a, w2t, b2f.reshape(hidden, 1))

    return edge_weight, edge_index


# speedup vs baseline: 1.4211x; 1.4211x over previous
import jax
import jax.numpy as jnp
from jax.experimental import pallas as pl

BN_EPS = 1e-5
COS_EPS = 1e-8
BLOCK_E = 4096


def _pae_block(xt_ref, w1t_ref, b1t_ref, w2t_ref, b2t_ref, o_ref):
    xt = xt_ref[...]
    w1t = w1t_ref[...]
    b1t = b1t_ref[...]
    w2t = w2t_ref[...]
    b2t = b2t_ref[...]
    in_dim = w1t.shape[1]
    x1t = xt[:in_dim, :]
    x2t = xt[in_dim:, :]
    a1 = jnp.maximum(jnp.dot(w1t, x1t, preferred_element_type=jnp.float32) + b1t, 0.0)
    a2 = jnp.maximum(jnp.dot(w1t, x2t, preferred_element_type=jnp.float32) + b1t, 0.0)
    h1 = jnp.dot(w2t, a1.astype(jnp.bfloat16), preferred_element_type=jnp.float32) + b2t
    h2 = jnp.dot(w2t, a2.astype(jnp.bfloat16), preferred_element_type=jnp.float32) + b2t
    s11 = jnp.sum(h1 * h1, axis=0)
    s22 = jnp.sum(h2 * h2, axis=0)
    s12 = jnp.sum(h1 * h2, axis=0)
    n1 = jnp.maximum(jnp.sqrt(s11), COS_EPS)
    n2 = jnp.maximum(jnp.sqrt(s22), COS_EPS)
    o_ref[...] = (s12 / (n1 * n2) + 1.0) * 0.5


def kernel(edge_index, edgenet_input, flag, W1, b1, gamma, beta,
           running_mean, running_var, W2, b2):
    n_edges, feat = edgenet_input.shape
    in_dim = feat // 2
    hidden = W1.shape[1]

    xt = edgenet_input.T.astype(jnp.bfloat16)
    scale = gamma * jax.lax.rsqrt(running_var + BN_EPS)
    w1t = W1.T.astype(jnp.bfloat16)
    w2t = (W2 * scale[:, None]).T.astype(jnp.bfloat16)
    b2f = b2 + (beta - running_mean * scale) @ W2

    edge_weight = pl.pallas_call(
        _pae_block,
        grid=(pl.cdiv(n_edges, BLOCK_E),),
        in_specs=[
            pl.BlockSpec((feat, BLOCK_E), lambda i: (0, i)),
            pl.BlockSpec((hidden, in_dim), lambda i: (0, 0)),
            pl.BlockSpec((hidden, 1), lambda i: (0, 0)),
            pl.BlockSpec((hidden, hidden), lambda i: (0, 0)),
            pl.BlockSpec((hidden, 1), lambda i: (0, 0)),
        ],
        out_specs=pl.BlockSpec((BLOCK_E,), lambda i: (i,)),
        out_shape=jax.ShapeDtypeStruct((n_edges,), jnp.float32),
    )(xt, w1t, b1.reshape(hidden, 1), w2t, b2f.reshape(hidden, 1))

    return edge_weight, edge_index
